# baseline (speedup 1.0000x reference)
import functools

import jax
import jax.numpy as jnp
from jax import lax
from jax.experimental import pallas as pl
from jax.experimental.pallas import tpu as pltpu
from jax.experimental.pallas import tpu_sc as plsc

_NC = 2
_NS = 16
_NW = _NC * _NS
_L = 16


def _table_body(emb_ref, w135_ref, b135_ref, w4_ref, b4_ref, t0_ref, t1_ref):
    e = emb_ref[:]
    h = jnp.dot(e, w135_ref[:], preferred_element_type=jnp.float32)
    h = h + b135_ref[:]
    col = lax.broadcasted_iota(jnp.int32, h.shape, 1)
    sig = jax.nn.sigmoid(h)
    act = jnp.where(col == 0, h, jnp.where(col == 1, sig, h * sig))
    t = jnp.dot(act, w4_ref[:], preferred_element_type=jnp.float32)
    t = t + b4_ref[:]
    t = t * jax.nn.sigmoid(t)
    t0_ref[:] = t[:, 0]
    t1_ref[:] = t[:, 1]


def _make_gather(batch: int, length: int, c_pad: int):
    rows_w = batch // _NW
    rows_p = 128
    n_pass = rows_w // rows_p
    n_grp = rows_p // _L
    mesh = plsc.VectorSubcoreMesh(core_axis_name="c", subcore_axis_name="s")

    @functools.partial(
        pl.kernel,
        mesh=mesh,
        out_type=jax.ShapeDtypeStruct((length, batch // 128, 2, 128),
                                      jnp.float32),
        scratch_types=[
            pltpu.VMEM((c_pad,), jnp.float32),
            pltpu.VMEM((c_pad,), jnp.float32),
            pltpu.VMEM((rows_p, length), jnp.int32),
            pltpu.VMEM((rows_p, length), jnp.int32),
            pltpu.VMEM((length, 2, 128), jnp.float32),
            pltpu.SemaphoreType.DMA,
            pltpu.SemaphoreType.DMA,
        ],
        compiler_params=pltpu.CompilerParams(needs_layout_passes=False, use_tc_tiling_on_sc=True),
    )
    def gather(t0_hbm, t1_hbm, idx_hbm, out_hbm, t0_v, t1_v, idx_a, idx_b,
               out_v, idx_sem, out_sem):
        wid = lax.axis_index("s") * _NC + lax.axis_index("c")
        pltpu.sync_copy(t0_hbm, t0_v)
        pltpu.sync_copy(t1_hbm, t1_v)
        iota = lax.iota(jnp.int32, _L)
        bufs = (idx_a, idx_b)

        def idx_start(p, buf):
            return pltpu.async_copy(
                idx_hbm.at[pl.ds(wid * rows_w + p * rows_p, rows_p), :],
                buf,
                idx_sem,
            )

        idx_start(0, bufs[0])
        out_dma = None
        for p in range(n_pass):
            pltpu.make_async_copy(
                idx_hbm.at[pl.ds(0, rows_p), :],
                bufs[p % 2],
                idx_sem,
            ).wait()
            if p + 1 < n_pass:
                idx_start(p + 1, bufs[(p + 1) % 2])
            if out_dma is not None:
                out_dma.wait()

            @plsc.parallel_loop(0, length, unroll=2)
            def l_body(l, buf=bufs[p % 2]):
                lvec = jnp.broadcast_to(l, (_L,)).astype(jnp.int32)
                for g in range(n_grp):
                    rvec = g * _L + iota
                    ids = plsc.load_gather(buf, [rvec, lvec])
                    v0 = plsc.load_gather(t0_v, [ids])
                    v1 = plsc.load_gather(t1_v, [ids])
                    out_v[l, 0, pl.ds(g * _L, _L)] = v0
                    out_v[l, 1, pl.ds(g * _L, _L)] = v1
            out_dma = pltpu.async_copy(
                out_v, out_hbm.at[:, wid * n_pass + p], out_sem
            )
        out_dma.wait()

    return gather


def kernel(inputs, emb, W1, b1, W2, b2, W3, b3, W4, b4):
    batch, length = inputs.shape
    num_classes, dim_emb = emb.shape
    c_pad = ((num_classes + 1023) // 1024) * 1024

    emb_p = jnp.pad(emb, ((0, c_pad - num_classes), (0, 0)))
    w135 = jnp.concatenate([W1, W2, W3], axis=1)
    b135 = jnp.concatenate([b1, b2, b3])[None, :]

    t0, t1 = pl.pallas_call(
        _table_body,
        out_shape=(
            jax.ShapeDtypeStruct((c_pad,), jnp.float32),
            jax.ShapeDtypeStruct((c_pad,), jnp.float32),
        ),
    )(emb_p, w135, b135, W4, b4[None, :])

    out4d = _make_gather(batch, length, c_pad)(t0, t1, inputs)
    return out4d.transpose(1, 3, 0, 2).reshape(batch, length, 2)


# speedup vs baseline: 1.5213x; 1.5213x over previous
import functools

import jax
import jax.numpy as jnp
from jax import lax
from jax.experimental import pallas as pl
from jax.experimental.pallas import tpu as pltpu
from jax.experimental.pallas import tpu_sc as plsc

_NC = 2
_NS = 16
_NW = _NC * _NS
_L = 16


def _table_body(emb_ref, w135_ref, b135_ref, w4_ref, b4_ref, t0_ref, t1_ref):
    e = emb_ref[:]
    h = jnp.dot(e, w135_ref[:], preferred_element_type=jnp.float32)
    h = h + b135_ref[:]
    col = lax.broadcasted_iota(jnp.int32, h.shape, 1)
    sig = jax.nn.sigmoid(h)
    act = jnp.where(col == 0, h, jnp.where(col == 1, sig, h * sig))
    t = jnp.dot(act, w4_ref[:], preferred_element_type=jnp.float32)
    t = t + b4_ref[:]
    t = t * jax.nn.sigmoid(t)
    t0_ref[:] = t[:, 0]
    t1_ref[:] = t[:, 1]


def _make_gather(batch: int, length: int, c_pad: int):
    rows_w = batch // _NW
    rows_p = 128
    n_pass = rows_w // rows_p
    n_grp = rows_p // _L
    mesh = plsc.VectorSubcoreMesh(core_axis_name="c", subcore_axis_name="s")

    @functools.partial(
        pl.kernel,
        mesh=mesh,
        out_type=jax.ShapeDtypeStruct((length, batch // 128, 2, 128),
                                      jnp.float32),
        scratch_types=[
            pltpu.VMEM((c_pad,), jnp.float32),
            pltpu.VMEM((c_pad,), jnp.float32),
            pltpu.VMEM((rows_p, length), jnp.int32),
            pltpu.VMEM((rows_p, length), jnp.int32),
            pltpu.VMEM((length, 2, 128), jnp.float32),
            pltpu.SemaphoreType.DMA,
            pltpu.SemaphoreType.DMA,
        ],
        compiler_params=pltpu.CompilerParams(needs_layout_passes=False, use_tc_tiling_on_sc=True),
    )
    def gather(t0_hbm, t1_hbm, idx_hbm, out_hbm, t0_v, t1_v, idx_a, idx_b,
               out_v, idx_sem, out_sem):
        wid = lax.axis_index("s") * _NC + lax.axis_index("c")
        pltpu.sync_copy(t0_hbm, t0_v)
        pltpu.sync_copy(t1_hbm, t1_v)
        iota = lax.iota(jnp.int32, _L)
        zeros = jnp.zeros((_L,), jnp.int32)
        ones = jnp.ones((_L,), jnp.int32)
        bufs = (idx_a, idx_b)

        def idx_start(p, buf):
            return pltpu.async_copy(
                idx_hbm.at[pl.ds(wid * rows_w + p * rows_p, rows_p), :],
                buf,
                idx_sem,
            )

        idx_start(0, bufs[0])
        out_dma = None
        for p in range(n_pass):
            pltpu.make_async_copy(
                idx_hbm.at[pl.ds(0, rows_p), :],
                bufs[p % 2],
                idx_sem,
            ).wait()
            if p + 1 < n_pass:
                idx_start(p + 1, bufs[(p + 1) % 2])
            if out_dma is not None:
                out_dma.wait()

            @plsc.parallel_loop(0, length, unroll=2)
            def l_body(l, buf=bufs[p % 2]):
                lv = l + iota
                lw = jnp.where(lv >= length, lv - length, lv)
                for g in range(n_grp):
                    rvec = g * _L + iota
                    ids = plsc.load_gather(buf, [rvec, lw])
                    v0 = plsc.load_gather(t0_v, [ids])
                    v1 = plsc.load_gather(t1_v, [ids])
                    plsc.store_scatter(out_v, [lw, zeros, rvec], v0)
                    plsc.store_scatter(out_v, [lw, ones, rvec], v1)
            out_dma = pltpu.async_copy(
                out_v, out_hbm.at[:, wid * n_pass + p], out_sem
            )
        out_dma.wait()

    return gather


def kernel(inputs, emb, W1, b1, W2, b2, W3, b3, W4, b4):
    batch, length = inputs.shape
    num_classes, dim_emb = emb.shape
    c_pad = ((num_classes + 1023) // 1024) * 1024

    emb_p = jnp.pad(emb, ((0, c_pad - num_classes), (0, 0)))
    w135 = jnp.concatenate([W1, W2, W3], axis=1)
    b135 = jnp.concatenate([b1, b2, b3])[None, :]

    t0, t1 = pl.pallas_call(
        _table_body,
        out_shape=(
            jax.ShapeDtypeStruct((c_pad,), jnp.float32),
            jax.ShapeDtypeStruct((c_pad,), jnp.float32),
        ),
    )(emb_p, w135, b135, W4, b4[None, :])

    out4d = _make_gather(batch, length, c_pad)(t0, t1, inputs)
    return out4d.transpose(1, 3, 0, 2).reshape(batch, length, 2)
